# baseline (speedup 1.0000x reference)
import jax
import jax.numpy as jnp
from jax import lax
from jax.experimental import pallas as pl
from jax.experimental.pallas import tpu as pltpu
from jax.experimental.pallas import tpu_sc as plsc

K = 5
W = 0.2
EPS = 1e-8

N = 100000
D = 128
B = 4096
C = 1000

ROWS_PER_STEP = 20000
N_STEPS = N // ROWS_PER_STEP
CE_ROWS = 2048
NEG_INF = float("-inf")


def _ce_kernel(p_ref, lab_ref, out_ref):
    i = pl.program_id(0)
    p = p_ref[...]
    lab = lab_ref[...]
    rowmax = jnp.max(p, axis=1, keepdims=True)
    lse = jnp.log(jnp.sum(jnp.exp(p - rowmax), axis=1, keepdims=True)) + rowmax
    cols = jax.lax.broadcasted_iota(jnp.int32, p.shape, 1)
    sel = jnp.sum(jnp.where(cols == lab, p, 0.0), axis=1, keepdims=True)
    nll_sum = jnp.sum(lse - sel)

    @pl.when(i == 0)
    def _init():
        out_ref[0, 0] = 0.0

    out_ref[0, 0] += nll_sum


def _cos_kernel(x_ref, tf_ref, cos_ref):
    x = x_ref[...]
    xn = x * jax.lax.rsqrt(jnp.maximum(jnp.sum(x * x), EPS * EPS))
    q = tf_ref[...]
    dims = (((1,), (1,)), ((), ()))
    raw = jax.lax.dot_general(xn, q, dims,
                              preferred_element_type=jnp.float32)
    ones = jnp.ones((1, D), dtype=jnp.float32)
    nrm2 = jax.lax.dot_general(ones, q * q, dims,
                               preferred_element_type=jnp.float32)
    inv = 1.0 / jnp.maximum(jnp.sqrt(nrm2), EPS)
    cos_ref[...] = (raw * inv)[:, None, :]


NW = 16
N_PAD = 100096
CHUNK = N_PAD // NW
N_CHUNKS = CHUNK // 16
L = 16


def _splat_max(m):
    lanes = lax.iota(jnp.int32, L)
    dnums = lax.GatherDimensionNumbers(
        offset_dims=(), collapsed_slice_dims=(0,), start_index_map=(0,))
    for sh in (1, 2, 4, 8):
        perm = jnp.bitwise_xor(lanes, sh)
        g = lax.gather(m, perm[:, None], dnums, (1,),
                       mode=lax.GatherScatterMode.PROMISE_IN_BOUNDS)
        m = jnp.maximum(m, g)
    return m


def _sc_select_kernel(cos_hbm, lab_hbm, xl_hbm, out_hbm, shared,
                      cos_v, lab_v, xl_v, stage_v, pub_v, merged_v, out_v,
                      sem1, sem2, sem3):
    wid = lax.axis_index("s")
    base = wid * CHUNK
    c1 = pltpu.async_copy(cos_hbm.at[pl.ds(base, CHUNK)], cos_v, sem1)
    c2 = pltpu.async_copy(lab_hbm.at[pl.ds(base, CHUNK)], lab_v, sem2)
    c3 = pltpu.async_copy(xl_hbm, xl_v, sem3)
    c1.wait()
    c2.wait()
    c3.wait()
    xl = xl_v[...]
    ninf = jnp.full((L,), NEG_INF, jnp.float32)

    def insert(tops, a):
        out = []
        for t in tops:
            out.append(jnp.maximum(t, a))
            a = jnp.minimum(t, a)
        return out

    for r in range(2 * K):
        stage_v[r] = ninf

    def body(i, carry):
        v = cos_v[pl.ds(i * L, L)]
        lab = lab_v[pl.ds(i * L, L)]
        pos = lab == xl
        a = jnp.where(pos, v, ninf)
        for r in range(K):
            t = stage_v[r]
            stage_v[r] = jnp.maximum(t, a)
            a = jnp.minimum(t, a)
        b = jnp.where(pos, ninf, 1.0 / v)
        for r in range(K, 2 * K):
            t = stage_v[r]
            stage_v[r] = jnp.maximum(t, b)
            b = jnp.minimum(t, b)
        return carry

    lax.fori_loop(0, N_CHUNKS, body, 0)
    for r in range(2 * K):
        pub_v[r] = stage_v[r]
    pltpu.sync_copy(pub_v, shared.at[wid])
    plsc.subcore_barrier()

    @pl.when(wid == 0)
    def _finalize():
        pltpu.sync_copy(shared, merged_v)

        def top5_sum(k_off, transform):
            tops = [ninf] * K
            for w in range(NW):
                for k in range(K):
                    tops = insert(tops, merged_v[w, k_off + k])
            acc = jnp.zeros((L,), jnp.float32)
            for _ in range(K):
                m = tops[0]
                for t in tops[1:]:
                    m = jnp.maximum(m, t)
                s = _splat_max(m)
                acc = acc + transform(s)
                tops = [jnp.where(t == s, ninf, t) for t in tops]
            return acc

        out_v[0] = top5_sum(0, lambda s: s)
        out_v[1] = top5_sum(K, lambda s: jnp.exp(1.0 / s))
        pltpu.sync_copy(out_v, out_hbm)


def _sc_select(cos_flat, labels, x_label):
    cos_pad = jnp.concatenate(
        [cos_flat, jnp.full((N_PAD - N,), -0.0, jnp.float32)])
    lab_pad = jnp.concatenate(
        [labels.astype(jnp.int32), jnp.full((N_PAD - N,), -1, jnp.int32)])
    xl16 = jnp.full((L,), x_label, jnp.int32)
    out, _ = pl.kernel(
        _sc_select_kernel,
        out_type=(jax.ShapeDtypeStruct((2, L), jnp.float32),
                  jax.ShapeDtypeStruct((NW, 2 * K, L), jnp.float32)),
        mesh=plsc.VectorSubcoreMesh(
            core_axis_name="c", subcore_axis_name="s", num_cores=1),
        scratch_types=[
            pltpu.VMEM((CHUNK,), jnp.float32),
            pltpu.VMEM((CHUNK,), jnp.int32),
            pltpu.VMEM((L,), jnp.int32),
            pltpu.VMEM((2 * K, L), jnp.float32),
            pltpu.VMEM((2 * K, L), jnp.float32),
            pltpu.VMEM((NW, 2 * K, L), jnp.float32),
            pltpu.VMEM((2, L), jnp.float32),
            pltpu.SemaphoreType.DMA,
            pltpu.SemaphoreType.DMA,
            pltpu.SemaphoreType.DMA,
        ],
    )(cos_pad, lab_pad, xl16)
    return out


def _select_kernel(cos_ref, lab_ref, xl_ref, ce_ref, out_ref):
    cos = cos_ref[...]
    lab = lab_ref[...]
    xl = xl_ref[0]
    pos = lab == xl

    ps = jnp.where(pos, cos, NEG_INF)
    pos_sum = jnp.float32(0.0)
    for _ in range(K):
        m = jnp.max(ps)
        pos_sum += m
        ps = jnp.where(ps >= m, NEG_INF, ps)

    ns = jnp.where(pos, NEG_INF, 1.0 / cos)
    den = jnp.float32(0.0)
    for _ in range(K):
        v = jnp.max(ns)
        den += jnp.exp(1.0 / v)
        ns = jnp.where(ns >= v, NEG_INF, ns)

    contrastive = (-1.0 / (2.0 * K)) * (pos_sum - K * jnp.log(den))
    ce = ce_ref[0] / jnp.float32(B)
    out_ref[0] = ce * (1.0 - W) + contrastive * W


def kernel(label, predict, x_feature, x_label, train_features, train_labels):
    ce_sum = pl.pallas_call(
        _ce_kernel,
        grid=(B // CE_ROWS,),
        in_specs=[
            pl.BlockSpec((CE_ROWS, C), lambda i: (i, 0)),
            pl.BlockSpec((CE_ROWS, 1), lambda i: (i, 0)),
        ],
        out_specs=pl.BlockSpec(memory_space=pltpu.SMEM),
        out_shape=jax.ShapeDtypeStruct((1, 1), jnp.float32),
    )(predict, label.astype(jnp.int32).reshape(B, 1))

    cos = pl.pallas_call(
        _cos_kernel,
        grid=(N_STEPS,),
        in_specs=[
            pl.BlockSpec((1, D), lambda i: (0, 0)),
            pl.BlockSpec((ROWS_PER_STEP, D), lambda i: (i, 0)),
        ],
        out_specs=pl.BlockSpec((1, 1, ROWS_PER_STEP), lambda i: (i, 0, 0)),
        out_shape=jax.ShapeDtypeStruct((N_STEPS, 1, ROWS_PER_STEP), jnp.float32),
    )(x_feature.reshape(1, D), train_features)

    sel = _sc_select(cos.reshape(N,), train_labels, x_label)
    pos_sum, den = sel[0, 0], sel[1, 0]
    contrastive = (-1.0 / (2.0 * K)) * (pos_sum - K * jnp.log(den))
    ce = ce_sum[0, 0] / jnp.float32(B)
    return ce * (1.0 - W) + contrastive * W


# speedup vs baseline: 1.0007x; 1.0007x over previous
import jax
import jax.numpy as jnp
from jax import lax
from jax.experimental import pallas as pl
from jax.experimental.pallas import tpu as pltpu
from jax.experimental.pallas import tpu_sc as plsc

K = 5
W = 0.2
EPS = 1e-8

N = 100000
D = 128
B = 4096
C = 1000

ROWS_PER_STEP = 20000
N_STEPS = N // ROWS_PER_STEP
CE_ROWS = 2048
NEG_INF = float("-inf")


def _ce_kernel(p_ref, lab_ref, out_ref):
    i = pl.program_id(0)
    p = p_ref[...]
    lab = lab_ref[...]
    rowmax = jnp.max(p, axis=1, keepdims=True)
    lse = jnp.log(jnp.sum(jnp.exp(p - rowmax), axis=1, keepdims=True)) + rowmax
    cols = jax.lax.broadcasted_iota(jnp.int32, p.shape, 1)
    sel = jnp.sum(jnp.where(cols == lab, p, 0.0), axis=1, keepdims=True)
    nll_sum = jnp.sum(lse - sel)

    @pl.when(i == 0)
    def _init():
        out_ref[0, 0] = 0.0

    out_ref[0, 0] += nll_sum


def _cos_kernel(x_ref, tf_ref, cos_ref):
    x = x_ref[...]
    xn = x * jax.lax.rsqrt(jnp.maximum(jnp.sum(x * x), EPS * EPS))
    q = tf_ref[...]
    dims = (((1,), (1,)), ((), ()))
    raw = jax.lax.dot_general(xn, q, dims,
                              preferred_element_type=jnp.float32)
    ones = jnp.ones((1, D), dtype=jnp.float32)
    nrm2 = jax.lax.dot_general(ones, q * q, dims,
                               preferred_element_type=jnp.float32)
    inv = 1.0 / jnp.maximum(jnp.sqrt(nrm2), EPS)
    cos_ref[...] = (raw * inv)[:, None, :]


NW = 16
N_PAD = 100096
CHUNK = N_PAD // NW
N_CHUNKS = CHUNK // 16
L = 16


def _splat_max(m):
    lanes = lax.iota(jnp.int32, L)
    dnums = lax.GatherDimensionNumbers(
        offset_dims=(), collapsed_slice_dims=(0,), start_index_map=(0,))
    for sh in (1, 2, 4, 8):
        perm = jnp.bitwise_xor(lanes, sh)
        g = lax.gather(m, perm[:, None], dnums, (1,),
                       mode=lax.GatherScatterMode.PROMISE_IN_BOUNDS)
        m = jnp.maximum(m, g)
    return m


def _sc_select_kernel(cos_hbm, lab_hbm, xl_hbm, out_hbm, shared,
                      cos_v, lab_v, xl_v, stage_v, pub_v, merged_v, out_v,
                      sem1, sem2, sem3):
    wid = lax.axis_index("s")
    base = wid * CHUNK
    c1 = pltpu.async_copy(cos_hbm.at[pl.ds(base, CHUNK)], cos_v, sem1)
    c2 = pltpu.async_copy(lab_hbm.at[pl.ds(base, CHUNK)], lab_v, sem2)
    c3 = pltpu.async_copy(xl_hbm, xl_v, sem3)
    c1.wait()
    c2.wait()
    c3.wait()
    xl = xl_v[...]
    ninf = jnp.full((L,), NEG_INF, jnp.float32)

    def insert(tops, a):
        out = []
        for t in tops:
            out.append(jnp.maximum(t, a))
            a = jnp.minimum(t, a)
        return out

    for r in range(2 * K):
        stage_v[r] = ninf

    def body(i, carry):
        v = cos_v[pl.ds(i * L, L)]
        lab = lab_v[pl.ds(i * L, L)]
        pos = lab == xl
        a = jnp.where(pos, v, ninf)
        for r in range(K):
            t = stage_v[r]
            stage_v[r] = jnp.maximum(t, a)
            a = jnp.minimum(t, a)
        b = jnp.where(pos, ninf, 1.0 / v)
        for r in range(K, 2 * K):
            t = stage_v[r]
            stage_v[r] = jnp.maximum(t, b)
            b = jnp.minimum(t, b)
        return carry

    lax.fori_loop(0, N_CHUNKS, body, 0)
    for r in range(2 * K):
        pub_v[r] = stage_v[r]
    pltpu.sync_copy(pub_v, shared.at[wid])
    plsc.subcore_barrier()

    @pl.when(wid == 0)
    def _finalize():
        pltpu.sync_copy(shared, merged_v)

        def top5_sum(k_off, transform):
            tops = [ninf] * K
            for w in range(NW):
                for k in range(K):
                    tops = insert(tops, merged_v[w, k_off + k])
            acc = jnp.zeros((L,), jnp.float32)
            for _ in range(K):
                m = tops[0]
                for t in tops[1:]:
                    m = jnp.maximum(m, t)
                s = _splat_max(m)
                acc = acc + transform(s)
                tops = [jnp.where(t == s, ninf, t) for t in tops]
            return acc

        out_v[0] = top5_sum(0, lambda s: s)
        out_v[1] = top5_sum(K, lambda s: jnp.exp(1.0 / s))
        pltpu.sync_copy(out_v, out_hbm)


def _sc_select(cos_flat, labels, x_label):
    cos_pad = jnp.concatenate(
        [cos_flat, jnp.full((N_PAD - N,), -0.0, jnp.float32)])
    lab_pad = jnp.concatenate(
        [labels.astype(jnp.int32), jnp.full((N_PAD - N,), -1, jnp.int32)])
    xl16 = jnp.full((L,), x_label, jnp.int32)
    out, _ = pl.kernel(
        _sc_select_kernel,
        out_type=(jax.ShapeDtypeStruct((2, L), jnp.float32),
                  jax.ShapeDtypeStruct((NW, 2 * K, L), jnp.float32)),
        mesh=plsc.VectorSubcoreMesh(
            core_axis_name="c", subcore_axis_name="s", num_cores=1),
        scratch_types=[
            pltpu.VMEM((CHUNK,), jnp.float32),
            pltpu.VMEM((CHUNK,), jnp.int32),
            pltpu.VMEM((L,), jnp.int32),
            pltpu.VMEM((2 * K, L), jnp.float32),
            pltpu.VMEM((2 * K, L), jnp.float32),
            pltpu.VMEM((NW, 2 * K, L), jnp.float32),
            pltpu.VMEM((2, L), jnp.float32),
            pltpu.SemaphoreType.DMA,
            pltpu.SemaphoreType.DMA,
            pltpu.SemaphoreType.DMA,
        ],
    )(cos_pad, lab_pad, xl16)
    return out


def _select_kernel(cos_ref, lab_ref, xl_ref, ce_ref, out_ref):
    cos = cos_ref[...]
    lab = lab_ref[...]
    xl = xl_ref[0]
    pos = lab == xl

    ps = jnp.where(pos, cos, NEG_INF)
    pos_sum = jnp.float32(0.0)
    for _ in range(K):
        m = jnp.max(ps)
        pos_sum += m
        ps = jnp.where(ps >= m, NEG_INF, ps)

    ns = jnp.where(pos, NEG_INF, 1.0 / cos)
    den = jnp.float32(0.0)
    for _ in range(K):
        v = jnp.max(ns)
        den += jnp.exp(1.0 / v)
        ns = jnp.where(ns >= v, NEG_INF, ns)

    contrastive = (-1.0 / (2.0 * K)) * (pos_sum - K * jnp.log(den))
    ce = ce_ref[0] / jnp.float32(B)
    out_ref[0] = ce * (1.0 - W) + contrastive * W


def kernel(label, predict, x_feature, x_label, train_features, train_labels):
    cos = pl.pallas_call(
        _cos_kernel,
        grid=(N_STEPS,),
        in_specs=[
            pl.BlockSpec((1, D), lambda i: (0, 0)),
            pl.BlockSpec((ROWS_PER_STEP, D), lambda i: (i, 0)),
        ],
        out_specs=pl.BlockSpec((1, 1, ROWS_PER_STEP), lambda i: (i, 0, 0)),
        out_shape=jax.ShapeDtypeStruct((N_STEPS, 1, ROWS_PER_STEP), jnp.float32),
    )(x_feature.reshape(1, D), train_features)

    sel = _sc_select(cos.reshape(N,), train_labels, x_label)

    ce_sum = pl.pallas_call(
        _ce_kernel,
        grid=(B // CE_ROWS,),
        in_specs=[
            pl.BlockSpec((CE_ROWS, C), lambda i: (i, 0)),
            pl.BlockSpec((CE_ROWS, 1), lambda i: (i, 0)),
        ],
        out_specs=pl.BlockSpec(memory_space=pltpu.SMEM),
        out_shape=jax.ShapeDtypeStruct((1, 1), jnp.float32),
    )(predict, label.astype(jnp.int32).reshape(B, 1))
    pos_sum, den = sel[0, 0], sel[1, 0]
    contrastive = (-1.0 / (2.0 * K)) * (pos_sum - K * jnp.log(den))
    ce = ce_sum[0, 0] / jnp.float32(B)
    return ce * (1.0 - W) + contrastive * W
